# baseline (speedup 1.0000x reference)
import jax
import jax.numpy as jnp
from jax import lax
from jax.experimental import pallas as pl
from jax.experimental.pallas import tpu as pltpu
from jax.experimental.pallas import tpu_sc as plsc

_F = 26
_V = 100000
_D = 64
_B = 4096
_L = 16

_UNROLL = 8

_NC = 2
_NS = 16
_NW = _NC * _NS
_DPW = _D // _NW

_CHUNKS = ((0, 25088), (25088, 24960), (50048, 24960), (75008, 24960))
_CMAX = 25088
_VMAIN = 99968
_TAIL = _V - _VMAIN


def _sc_body(xt_hbm, tt_hbm, tail_hbm, out_hbm, xbuf, bufa, bufb, tailbuf,
             acc, sema, semb, semt):
    cid = lax.axis_index("c")
    sid = lax.axis_index("s")
    wid = sid * _NC + cid
    d0 = wid * _DPW

    zero = jnp.zeros((_L,), jnp.float32)

    def zacc(j, carry):
        p = pl.multiple_of(j * _L, _L)
        acc[0, pl.ds(p, _L)] = zero
        acc[1, pl.ds(p, _L)] = zero
        return carry

    lax.fori_loop(0, _B // _L, zacc, 0)

    bufs = (bufa, bufb, bufa, bufb)
    sems = (sema, semb, sema, semb)

    def issue(f, k):
        c0, csz = _CHUNKS[k]
        row0 = f * _D + d0
        pltpu.async_copy(
            tt_hbm.at[pl.ds(row0, _DPW), pl.ds(c0, csz)],
            bufs[k].at[:, pl.ds(0, csz)], sems[k])
        if k == len(_CHUNKS) - 1:
            pltpu.async_copy(tail_hbm.at[pl.ds(row0, _DPW)], tailbuf, semt)

    issue(0, 0)
    issue(0, 1)

    dvecs = tuple(jnp.full((_L,), dd, jnp.int32) for dd in range(_DPW))

    def field(f, carry):
        pltpu.sync_copy(xt_hbm.at[f], xbuf)
        row0 = f * _D + d0
        for k, (c0, csz) in enumerate(_CHUNKS):
            buf = bufs[k]
            last = k == len(_CHUNKS) - 1
            pltpu.make_async_copy(
                tt_hbm.at[pl.ds(row0, _DPW), pl.ds(c0, csz)],
                buf.at[:, pl.ds(0, csz)], sems[k]).wait()
            span = csz
            if last:
                pltpu.make_async_copy(
                    tail_hbm.at[pl.ds(row0, _DPW)], tailbuf, semt).wait()
                for dd in range(_DPW):
                    buf[dd, pl.ds(csz, _L)] = tailbuf[dd, pl.ds(0, _L)]
                    buf[dd, pl.ds(csz + _L, _L)] = tailbuf[dd, pl.ds(_L, _L)]
                span = csz + _TAIL

            def bgroup(j, carry2, buf=buf, c0=c0, span=span):
                base = pl.multiple_of(j * (_L * _UNROLL), _L)
                for u in range(_UNROLL):
                    p = base + u * _L
                    idx = xbuf[pl.ds(p, _L)] - c0
                    m = plsc.bitcast(idx, jnp.uint32) < jnp.uint32(span)
                    for dd in range(_DPW):
                        vals = plsc.load_gather(buf, [dvecs[dd], idx], mask=m)
                        vals = jnp.where(m, vals, 0.0)
                        plsc.addupdate(acc.at[dd, pl.ds(p, _L)], vals)
                return carry2

            lax.fori_loop(0, _B // (_L * _UNROLL), bgroup, 0)

            if k + 2 < len(_CHUNKS):
                issue(f, k + 2)
            else:
                @pl.when(f < _F - 1)
                def _():
                    issue(f + 1, k + 2 - len(_CHUNKS))
        return carry

    lax.fori_loop(0, _F, field, 0)

    for dd in range(_DPW):
        pltpu.sync_copy(acc.at[dd], out_hbm.at[d0 + dd])


def kernel(x, tables):
    xt = x.T.astype(jnp.int32)
    tt = tables.transpose(0, 2, 1).reshape(_F * _D, _V)
    tail = tables[:, _VMAIN:, :].transpose(0, 2, 1).reshape(_F * _D, _TAIL)
    mesh = plsc.VectorSubcoreMesh(core_axis_name="c", subcore_axis_name="s")
    run = pl.kernel(
        _sc_body,
        mesh=mesh,
        compiler_params=pltpu.CompilerParams(needs_layout_passes=False),
        out_type=jax.ShapeDtypeStruct((_D, _B), jnp.float32),
        scratch_types=[
            pltpu.VMEM((_B,), jnp.int32),
            pltpu.VMEM((_DPW, _CMAX + _TAIL), jnp.float32),
            pltpu.VMEM((_DPW, _CMAX + _TAIL), jnp.float32),
            pltpu.VMEM((_DPW, _TAIL), jnp.float32),
            pltpu.VMEM((_DPW, _B), jnp.float32),
            pltpu.SemaphoreType.DMA,
            pltpu.SemaphoreType.DMA,
            pltpu.SemaphoreType.DMA,
        ],
    )
    out = run(xt, tt, tail)
    return out.T


# speedup vs baseline: 1.3377x; 1.3377x over previous
import jax
import jax.numpy as jnp
from jax import lax
from jax.experimental import pallas as pl
from jax.experimental.pallas import tpu as pltpu
from jax.experimental.pallas import tpu_sc as plsc

_F = 26
_V = 100000
_D = 64
_B = 4096
_L = 16

_UNROLL = 8

_NC = 2
_NS = 16
_NW = _NC * _NS
_DPW = _D // _NW

_C0 = 50048
_C1 = 49920
_VMAIN = _C0 + _C1
_TAIL = _V - _VMAIN


def _sc_body(xt_hbm, tt_hbm, tail_hbm, out_hbm, xbuf, bufa, bufb, tailbuf,
             acc, sema, semb, semt):
    cid = lax.axis_index("c")
    sid = lax.axis_index("s")
    wid = sid * _NC + cid
    d0 = wid * _DPW

    zero = jnp.zeros((_L,), jnp.float32)

    def zacc(j, carry):
        p = pl.multiple_of(j * _L, _L)
        acc[0, pl.ds(p, _L)] = zero
        acc[1, pl.ds(p, _L)] = zero
        return carry

    lax.fori_loop(0, _B // _L, zacc, 0)

    def issue_a(row):
        return pltpu.async_copy(
            tt_hbm.at[row, pl.ds(0, _C0)], bufa, sema)

    def issue_b(row):
        pltpu.async_copy(
            tt_hbm.at[row, pl.ds(_C0, _C1)], bufb.at[pl.ds(0, _C1)], semb)
        pltpu.async_copy(tail_hbm.at[row], tailbuf, semt)

    issue_a(d0)
    issue_b(d0)

    def field(f, carry):
        pltpu.sync_copy(xt_hbm.at[f], xbuf)
        for dd in range(_DPW):
            row = f * _D + d0 + dd

            pltpu.make_async_copy(
                tt_hbm.at[row, pl.ds(0, _C0)], bufa, sema).wait()

            def bgroup_a(j, carry2):
                base = pl.multiple_of(j * (_L * _UNROLL), _L)
                for u in range(_UNROLL):
                    p = base + u * _L
                    idx = xbuf[pl.ds(p, _L)]
                    m = idx < _C0
                    vals = plsc.load_gather(bufa, [idx], mask=m)
                    vals = jnp.where(m, vals, 0.0)
                    plsc.addupdate(acc.at[dd, pl.ds(p, _L)], vals)
                return carry2

            lax.fori_loop(0, _B // (_L * _UNROLL), bgroup_a, 0)

            if dd + 1 < _DPW:
                issue_a(row + 1)
            else:
                @pl.when(f < _F - 1)
                def _():
                    issue_a((f + 1) * _D + d0)

            pltpu.make_async_copy(
                tt_hbm.at[row, pl.ds(_C0, _C1)], bufb.at[pl.ds(0, _C1)],
                semb).wait()
            pltpu.make_async_copy(tail_hbm.at[row], tailbuf, semt).wait()
            bufb[pl.ds(_C1, _L)] = tailbuf[pl.ds(0, _L)]
            bufb[pl.ds(_C1 + _L, _L)] = tailbuf[pl.ds(_L, _L)]

            def bgroup_b(j, carry2):
                base = pl.multiple_of(j * (_L * _UNROLL), _L)
                for u in range(_UNROLL):
                    p = base + u * _L
                    idx = xbuf[pl.ds(p, _L)] - _C0
                    m = plsc.bitcast(idx, jnp.uint32) < jnp.uint32(
                        _C1 + _TAIL)
                    vals = plsc.load_gather(bufb, [idx], mask=m)
                    vals = jnp.where(m, vals, 0.0)
                    plsc.addupdate(acc.at[dd, pl.ds(p, _L)], vals)
                return carry2

            lax.fori_loop(0, _B // (_L * _UNROLL), bgroup_b, 0)

            if dd + 1 < _DPW:
                issue_b(row + 1)
            else:
                @pl.when(f < _F - 1)
                def _():
                    issue_b((f + 1) * _D + d0)
        return carry

    lax.fori_loop(0, _F, field, 0)

    for dd in range(_DPW):
        pltpu.sync_copy(acc.at[dd], out_hbm.at[d0 + dd])


def kernel(x, tables):
    xt = x.T.astype(jnp.int32)
    tt = tables.transpose(0, 2, 1).reshape(_F * _D, _V)
    tail = tables[:, _VMAIN:, :].transpose(0, 2, 1).reshape(_F * _D, _TAIL)
    mesh = plsc.VectorSubcoreMesh(core_axis_name="c", subcore_axis_name="s")
    run = pl.kernel(
        _sc_body,
        mesh=mesh,
        compiler_params=pltpu.CompilerParams(needs_layout_passes=False),
        out_type=jax.ShapeDtypeStruct((_D, _B), jnp.float32),
        scratch_types=[
            pltpu.VMEM((_B,), jnp.int32),
            pltpu.VMEM((_C0,), jnp.float32),
            pltpu.VMEM((_C1 + _TAIL,), jnp.float32),
            pltpu.VMEM((_TAIL,), jnp.float32),
            pltpu.VMEM((_DPW, _B), jnp.float32),
            pltpu.SemaphoreType.DMA,
            pltpu.SemaphoreType.DMA,
            pltpu.SemaphoreType.DMA,
        ],
    )
    out = run(xt, tt, tail)
    return out.T


# speedup vs baseline: 1.4802x; 1.1066x over previous
import jax
import jax.numpy as jnp
from jax import lax
from jax.experimental import pallas as pl
from jax.experimental.pallas import tpu as pltpu
from jax.experimental.pallas import tpu_sc as plsc

_F = 26
_V = 100000
_D = 64
_B = 4096
_L = 16

_UNROLL = 8

_NC = 2
_NS = 16
_NW = _NC * _NS
_DPW = _D // _NW

_C0 = 50048
_C1 = 49920
_VMAIN = _C0 + _C1
_TAIL = _V - _VMAIN


def _sc_body(xt_hbm, tt_hbm, tail_hbm, out_hbm, xbufs, bufa, bufb, tailbuf,
             acc, sema, semb, semt, semx):
    cid = lax.axis_index("c")
    sid = lax.axis_index("s")
    wid = sid * _NC + cid
    d0 = wid * _DPW

    pltpu.async_copy(xt_hbm.at[0], xbufs.at[0], semx)

    zero = jnp.zeros((_L,), jnp.float32)

    def zacc(j, carry):
        p = pl.multiple_of(j * _L, _L)
        acc[0, pl.ds(p, _L)] = zero
        acc[1, pl.ds(p, _L)] = zero
        return carry

    lax.fori_loop(0, _B // _L, zacc, 0)

    def issue_a(row):
        return pltpu.async_copy(
            tt_hbm.at[row, pl.ds(0, _C0)], bufa, sema)

    def issue_b(row):
        pltpu.async_copy(
            tt_hbm.at[row, pl.ds(_C0, _C1)], bufb.at[pl.ds(0, _C1)], semb)
        pltpu.async_copy(tail_hbm.at[row], tailbuf, semt)

    issue_a(d0)
    issue_b(d0)

    def field_pair(jf, carry):
        for f2 in range(2):
            f = jf * 2 + f2
            pltpu.make_async_copy(xt_hbm.at[f], xbufs.at[f2], semx).wait()
            if f2 == 0:
                pltpu.async_copy(xt_hbm.at[f + 1], xbufs.at[1], semx)
            else:
                @pl.when(jf < _F // 2 - 1)
                def _():
                    pltpu.async_copy(xt_hbm.at[f + 1], xbufs.at[0], semx)
            for dd in range(_DPW):
                row = f * _D + d0 + dd

                pltpu.make_async_copy(
                    tt_hbm.at[row, pl.ds(0, _C0)], bufa, sema).wait()

                def bgroup_a(j, carry2, f2=f2, dd=dd):
                    base = pl.multiple_of(j * (_L * _UNROLL), _L)
                    for u in range(_UNROLL):
                        p = base + u * _L
                        idx = xbufs[f2, pl.ds(p, _L)]
                        m = idx < _C0
                        vals = plsc.load_gather(bufa, [idx], mask=m)
                        vals = jnp.where(m, vals, 0.0)
                        plsc.addupdate(acc.at[dd, pl.ds(p, _L)], vals)
                    return carry2

                lax.fori_loop(0, _B // (_L * _UNROLL), bgroup_a, 0)

                if dd + 1 < _DPW:
                    issue_a(row + 1)
                else:
                    @pl.when(f < _F - 1)
                    def _():
                        issue_a((f + 1) * _D + d0)

                pltpu.make_async_copy(
                    tt_hbm.at[row, pl.ds(_C0, _C1)], bufb.at[pl.ds(0, _C1)],
                    semb).wait()
                pltpu.make_async_copy(tail_hbm.at[row], tailbuf, semt).wait()
                bufb[pl.ds(_C1, _L)] = tailbuf[pl.ds(0, _L)]
                bufb[pl.ds(_C1 + _L, _L)] = tailbuf[pl.ds(_L, _L)]

                def bgroup_b(j, carry2, f2=f2, dd=dd):
                    base = pl.multiple_of(j * (_L * _UNROLL), _L)
                    for u in range(_UNROLL):
                        p = base + u * _L
                        idx = xbufs[f2, pl.ds(p, _L)] - _C0
                        m = plsc.bitcast(idx, jnp.uint32) < jnp.uint32(
                            _C1 + _TAIL)
                        vals = plsc.load_gather(bufb, [idx], mask=m)
                        vals = jnp.where(m, vals, 0.0)
                        plsc.addupdate(acc.at[dd, pl.ds(p, _L)], vals)
                    return carry2

                lax.fori_loop(0, _B // (_L * _UNROLL), bgroup_b, 0)

                if dd + 1 < _DPW:
                    issue_b(row + 1)
                else:
                    @pl.when(f < _F - 1)
                    def _():
                        issue_b((f + 1) * _D + d0)
        return carry

    lax.fori_loop(0, _F // 2, field_pair, 0)

    for dd in range(_DPW):
        pltpu.sync_copy(acc.at[dd], out_hbm.at[d0 + dd])


def kernel(x, tables):
    xt = x.T.astype(jnp.int32)
    tt = tables.transpose(0, 2, 1).reshape(_F * _D, _V)
    tail = tables[:, _VMAIN:, :].transpose(0, 2, 1).reshape(_F * _D, _TAIL)
    mesh = plsc.VectorSubcoreMesh(core_axis_name="c", subcore_axis_name="s")
    run = pl.kernel(
        _sc_body,
        mesh=mesh,
        compiler_params=pltpu.CompilerParams(needs_layout_passes=False),
        out_type=jax.ShapeDtypeStruct((_D, _B), jnp.float32),
        scratch_types=[
            pltpu.VMEM((2, _B), jnp.int32),
            pltpu.VMEM((_C0,), jnp.float32),
            pltpu.VMEM((_C1 + _TAIL,), jnp.float32),
            pltpu.VMEM((_TAIL,), jnp.float32),
            pltpu.VMEM((_DPW, _B), jnp.float32),
            pltpu.SemaphoreType.DMA,
            pltpu.SemaphoreType.DMA,
            pltpu.SemaphoreType.DMA,
            pltpu.SemaphoreType.DMA,
        ],
    )
    out = run(xt, tt, tail)
    return out.T


# speedup vs baseline: 1.5210x; 1.0275x over previous
import jax
import jax.numpy as jnp
from jax import lax
from jax.experimental import pallas as pl
from jax.experimental.pallas import tpu as pltpu
from jax.experimental.pallas import tpu_sc as plsc

_F = 26
_V = 100000
_D = 64
_B = 4096
_L = 16

_UNROLL = 8

_NC = 2
_NS = 16
_NW = _NC * _NS
_DPW = _D // _NW

_CHUNKS = ((0, 25088), (25088, 24960), (50048, 24960), (75008, 24960))
_CMAX = 25088
_VMAIN = 99968
_TAIL = _V - _VMAIN


def _sc_body(xt_hbm, tt_hbm, tail_hbm, out_hbm, xbufs, bufa, bufb, tailbuf,
             acc, sema, semb, semt, semx):
    cid = lax.axis_index("c")
    sid = lax.axis_index("s")
    wid = sid * _NC + cid
    d0 = wid * _DPW

    pltpu.async_copy(xt_hbm.at[0], xbufs.at[0], semx)

    zero = jnp.zeros((_L,), jnp.float32)

    def zacc(j, carry):
        p = pl.multiple_of(j * _L, _L)
        acc[0, pl.ds(p, _L)] = zero
        acc[1, pl.ds(p, _L)] = zero
        return carry

    lax.fori_loop(0, _B // _L, zacc, 0)

    bufs = (bufa, bufb, bufa, bufb)
    sems = (sema, semb, sema, semb)
    dvecs = tuple(jnp.full((_L,), dd, jnp.int32) for dd in range(_DPW))

    def issue(f, k):
        c0, csz = _CHUNKS[k]
        row0 = f * _D + d0
        pltpu.async_copy(
            tt_hbm.at[pl.ds(row0, _DPW), pl.ds(c0, csz)],
            bufs[k].at[:, pl.ds(0, csz)], sems[k])
        if k == len(_CHUNKS) - 1:
            pltpu.async_copy(tail_hbm.at[pl.ds(row0, _DPW)], tailbuf, semt)

    issue(0, 0)
    issue(0, 1)

    def field_pair(jf, carry):
        for f2 in range(2):
            f = jf * 2 + f2
            pltpu.make_async_copy(xt_hbm.at[f], xbufs.at[f2], semx).wait()
            if f2 == 0:
                pltpu.async_copy(xt_hbm.at[f + 1], xbufs.at[1], semx)
            else:
                @pl.when(jf < _F // 2 - 1)
                def _():
                    pltpu.async_copy(xt_hbm.at[f + 1], xbufs.at[0], semx)

            row0 = f * _D + d0
            for k, (c0, csz) in enumerate(_CHUNKS):
                buf = bufs[k]
                last = k == len(_CHUNKS) - 1
                pltpu.make_async_copy(
                    tt_hbm.at[pl.ds(row0, _DPW), pl.ds(c0, csz)],
                    buf.at[:, pl.ds(0, csz)], sems[k]).wait()
                span = csz
                if last:
                    pltpu.make_async_copy(
                        tail_hbm.at[pl.ds(row0, _DPW)], tailbuf, semt).wait()
                    for dd in range(_DPW):
                        buf[dd, pl.ds(csz, _L)] = tailbuf[dd, pl.ds(0, _L)]
                        buf[dd, pl.ds(csz + _L, _L)] = (
                            tailbuf[dd, pl.ds(_L, _L)])
                    span = csz + _TAIL

                def bgroup(j, carry2, buf=buf, c0=c0, span=span, f2=f2):
                    base = pl.multiple_of(j * (_L * _UNROLL), _L)
                    for u in range(_UNROLL):
                        p = base + u * _L
                        if c0:
                            idx = xbufs[f2, pl.ds(p, _L)] - c0
                            m = plsc.bitcast(idx, jnp.uint32) < jnp.uint32(
                                span)
                        else:
                            idx = xbufs[f2, pl.ds(p, _L)]
                            m = idx < span
                        for dd in range(_DPW):
                            vals = plsc.load_gather(
                                buf, [dvecs[dd], idx], mask=m)
                            vals = jnp.where(m, vals, 0.0)
                            plsc.addupdate(acc.at[dd, pl.ds(p, _L)], vals)
                    return carry2

                lax.fori_loop(0, _B // (_L * _UNROLL), bgroup, 0)

                if k + 2 < len(_CHUNKS):
                    issue(f, k + 2)
                else:
                    @pl.when(f < _F - 1)
                    def _():
                        issue(f + 1, k + 2 - len(_CHUNKS))
        return carry

    lax.fori_loop(0, _F // 2, field_pair, 0)

    for dd in range(_DPW):
        pltpu.sync_copy(acc.at[dd], out_hbm.at[d0 + dd])


def kernel(x, tables):
    xt = x.T.astype(jnp.int32)
    tt = tables.transpose(0, 2, 1).reshape(_F * _D, _V)
    tail = tables[:, _VMAIN:, :].transpose(0, 2, 1).reshape(_F * _D, _TAIL)
    mesh = plsc.VectorSubcoreMesh(core_axis_name="c", subcore_axis_name="s")
    run = pl.kernel(
        _sc_body,
        mesh=mesh,
        compiler_params=pltpu.CompilerParams(needs_layout_passes=False),
        out_type=jax.ShapeDtypeStruct((_D, _B), jnp.float32),
        scratch_types=[
            pltpu.VMEM((2, _B), jnp.int32),
            pltpu.VMEM((_DPW, _CMAX + _TAIL), jnp.float32),
            pltpu.VMEM((_DPW, _CMAX + _TAIL), jnp.float32),
            pltpu.VMEM((_DPW, _TAIL), jnp.float32),
            pltpu.VMEM((_DPW, _B), jnp.float32),
            pltpu.SemaphoreType.DMA,
            pltpu.SemaphoreType.DMA,
            pltpu.SemaphoreType.DMA,
            pltpu.SemaphoreType.DMA,
        ],
    )
    out = run(xt, tt, tail)
    return out.T
